# Initial kernel scaffold; baseline (speedup 1.0000x reference)
import jax
import jax.numpy as jnp
from jax.experimental import pallas as pl


def kernel(x, edge_index, W, b, gamma, beta, running_mean, running_var):
    raise NotImplementedError("write your pallas kernel here")


# speedup vs baseline: 22.2738x; 22.2738x over previous
import functools

import jax
import jax.numpy as jnp
from jax import lax
from jax.experimental import pallas as pl
from jax.experimental.pallas import tpu as pltpu
from jax.experimental.pallas import tpu_sc as plsc

_N = 10000
_E = 320000
_D = 128
_NC = 2
_NS = 16
_NW = _NC * _NS
_K = 80
_C = _E // (_NW * _K)
_NPAD = 10240
_BAND = _NPAD // _NS
_EPS = 1e-5
_RB = 2000
_G = _N // _RB
_WD = 128


def _sc_mesh():
    return plsc.VectorSubcoreMesh(
        core_axis_name="c", subcore_axis_name="s",
        num_cores=_NC, num_subcores=_NS)


def _deg_pass(dst3, ones_col, zeros_col):
    wd = ones_col.shape[1]
    @functools.partial(
        pl.kernel,
        out_type=jax.ShapeDtypeStruct((_NC, _NPAD, wd), jnp.float32),
        mesh=_sc_mesh(),
        scratch_types=[
            pltpu.VMEM((_C, _K), jnp.int32),
            pltpu.VMEM((_K, wd), jnp.float32),
            pltpu.VMEM_SHARED((_NPAD, wd), jnp.float32),
        ],
    )
    def k(dst_hbm, ones_hbm, zeros_hbm, out_hbm, idx_v, ones_v, deg_sh):
        c = lax.axis_index("c")
        s = lax.axis_index("s")
        wid = c * _NS + s
        pltpu.sync_copy(zeros_hbm.at[pl.ds(s * _BAND, _BAND)],
                        deg_sh.at[pl.ds(s * _BAND, _BAND)])
        pltpu.sync_copy(dst_hbm.at[wid], idx_v)
        pltpu.sync_copy(ones_hbm, ones_v)
        plsc.subcore_barrier()

        def body(j, carry):
            pltpu.sync_copy(ones_v, deg_sh.at[idx_v.at[j]], add=True)
            return carry

        lax.fori_loop(0, _C, body, 0)
        plsc.subcore_barrier()
        pltpu.sync_copy(deg_sh.at[pl.ds(s * _BAND, _BAND)],
                        out_hbm.at[c, pl.ds(s * _BAND, _BAND)])

    return k(dst3, ones_col, zeros_col)


def _scatter_pass(h2, src3, dst3, zeros_nd):
    @functools.partial(
        pl.kernel,
        out_type=jax.ShapeDtypeStruct((_NC, _NPAD, _D), jnp.float32),
        mesh=_sc_mesh(),
        scratch_types=[
            pltpu.VMEM((_C, _K), jnp.int32),
            pltpu.VMEM((_C, _K), jnp.int32),
            pltpu.VMEM((_K, _D), jnp.float32),
            pltpu.VMEM_SHARED((_NPAD, _D), jnp.float32),
            pltpu.SemaphoreType.DMA,
        ],
    )
    def k(h2_hbm, src_hbm, dst_hbm, zeros_hbm, out_hbm,
          sidx_v, didx_v, rows_v, acc_sh, sem):
        c = lax.axis_index("c")
        s = lax.axis_index("s")
        wid = c * _NS + s
        pltpu.sync_copy(zeros_hbm.at[pl.ds(s * _BAND, _BAND)],
                        acc_sh.at[pl.ds(s * _BAND, _BAND)])
        pltpu.sync_copy(src_hbm.at[wid], sidx_v)
        pltpu.sync_copy(dst_hbm.at[wid], didx_v)
        plsc.subcore_barrier()

        def body(j, carry):
            pltpu.async_copy(h2_hbm.at[sidx_v.at[j]], rows_v, sem).wait()
            pltpu.sync_copy(rows_v, acc_sh.at[didx_v.at[j]], add=True)
            return carry

        lax.fori_loop(0, _C, body, 0)
        plsc.subcore_barrier()
        pltpu.sync_copy(acc_sh.at[pl.ds(s * _BAND, _BAND)],
                        out_hbm.at[c, pl.ds(s * _BAND, _BAND)])

    return k(h2, src3, dst3, zeros_nd)


def _tc_a(x, W, gamma2, var2, d0, d1):
    def f(x_ref, w_ref, g_ref, v_ref, d0_ref, d1_ref, h2_ref, dinv_ref):
        sca = g_ref[...] * lax.rsqrt(v_ref[...] + _EPS)
        h = jnp.dot(x_ref[...], w_ref[...] * sca,
                    preferred_element_type=jnp.float32)
        dinv = lax.rsqrt(d0_ref[...] + d1_ref[...] + 1.0)
        h2_ref[...] = h * dinv
        dinv_ref[...] = dinv

    row = lambda i: (i, 0)
    rep = lambda i: (0, 0)
    return pl.pallas_call(
        f,
        grid=(_G,),
        in_specs=[
            pl.BlockSpec((_RB, _D), row),
            pl.BlockSpec((_D, _D), rep),
            pl.BlockSpec((1, _D), rep),
            pl.BlockSpec((1, _D), rep),
            pl.BlockSpec((_RB, 1), row),
            pl.BlockSpec((_RB, 1), row),
        ],
        out_specs=[
            pl.BlockSpec((_RB, _D), row),
            pl.BlockSpec((_RB, 1), row),
        ],
        out_shape=[
            jax.ShapeDtypeStruct((_N, _D), jnp.float32),
            jax.ShapeDtypeStruct((_N, 1), jnp.float32),
        ],
    )(x, W, gamma2, var2, d0, d1)


def _tc_b(p0, p1, h2, dinv, b2, mean2, beta2, gamma2, var2):
    def f(p0_ref, p1_ref, h2_ref, dinv_ref, b_ref, m_ref, bt_ref,
          g_ref, v_ref, out_ref):
        sca = g_ref[...] * lax.rsqrt(v_ref[...] + _EPS)
        t = (b_ref[...] - m_ref[...]) * sca + bt_ref[...]
        tot = p0_ref[...] + p1_ref[...] + h2_ref[...]
        out_ref[...] = jnp.maximum(tot * dinv_ref[...] + t, 0.0)

    row = lambda i: (i, 0)
    rep = lambda i: (0, 0)
    return pl.pallas_call(
        f,
        grid=(_G,),
        in_specs=[
            pl.BlockSpec((_RB, _D), row),
            pl.BlockSpec((_RB, _D), row),
            pl.BlockSpec((_RB, _D), row),
            pl.BlockSpec((_RB, 1), row),
            pl.BlockSpec((1, _D), rep),
            pl.BlockSpec((1, _D), rep),
            pl.BlockSpec((1, _D), rep),
            pl.BlockSpec((1, _D), rep),
            pl.BlockSpec((1, _D), rep),
        ],
        out_specs=pl.BlockSpec((_RB, _D), row),
        out_shape=jax.ShapeDtypeStruct((_N, _D), jnp.float32),
    )(p0, p1, h2, dinv, b2, mean2, beta2, gamma2, var2)


def kernel(x, edge_index, W, b, gamma, beta, running_mean, running_var):
    src3 = edge_index[0].astype(jnp.int32).reshape(_NW, _C, _K)
    dst3 = edge_index[1].astype(jnp.int32).reshape(_NW, _C, _K)
    ones_col = jnp.ones((_K, _WD), jnp.float32)
    zeros_nd = jnp.zeros((_NPAD, _D), jnp.float32)

    g2 = gamma.reshape(1, _D)
    v2 = running_var.reshape(1, _D)
    b2 = b.reshape(1, _D)
    m2 = running_mean.reshape(1, _D)
    bt2 = beta.reshape(1, _D)

    degp = _deg_pass(dst3, ones_col, zeros_nd)
    d0 = degp[0, :_N, 0:1]
    d1 = degp[1, :_N, 0:1]
    h2, dinv = _tc_a(x, W, g2, v2, d0, d1)
    acc = _scatter_pass(h2, src3, dst3, zeros_nd)
    p0 = acc[0, :_N]
    p1 = acc[1, :_N]
    return _tc_b(p0, p1, h2, dinv, b2, m2, bt2, g2, v2)


# speedup vs baseline: 29.7252x; 1.3345x over previous
import functools

import jax
import jax.numpy as jnp
from jax import lax
from jax.experimental import pallas as pl
from jax.experimental.pallas import tpu as pltpu
from jax.experimental.pallas import tpu_sc as plsc

_N = 10000
_E = 320000
_D = 128
_NC = 2
_NS = 16
_NW = _NC * _NS
_K = 100
_C = _E // (_NW * _K)
_WCH = 10
_NWIN = _C // _WCH
_NPAD = 10240
_BAND = _NPAD // _NS
_EPS = 1e-5
_RB = 2000
_G = _N // _RB
_WD = 128


def _sc_mesh():
    return plsc.VectorSubcoreMesh(
        core_axis_name="c", subcore_axis_name="s",
        num_cores=_NC, num_subcores=_NS)


def _deg_pass(dst3, ones_col, zeros_col):
    wd = ones_col.shape[1]
    @functools.partial(
        pl.kernel,
        out_type=jax.ShapeDtypeStruct((_NC, _NPAD, wd), jnp.float32),
        mesh=_sc_mesh(),
        scratch_types=[
            pltpu.VMEM((_C, _K), jnp.int32),
            pltpu.VMEM((_K, wd), jnp.float32),
            pltpu.VMEM_SHARED((_NPAD, wd), jnp.float32),
        ],
    )
    def k(dst_hbm, ones_hbm, zeros_hbm, out_hbm, idx_v, ones_v, deg_sh):
        c = lax.axis_index("c")
        s = lax.axis_index("s")
        wid = c * _NS + s
        pltpu.sync_copy(zeros_hbm.at[pl.ds(s * _BAND, _BAND)],
                        deg_sh.at[pl.ds(s * _BAND, _BAND)])
        pltpu.sync_copy(dst_hbm.at[wid], idx_v)
        pltpu.sync_copy(ones_hbm, ones_v)
        plsc.subcore_barrier()

        def body(j, carry):
            pltpu.sync_copy(ones_v, deg_sh.at[idx_v.at[j]], add=True)
            return carry

        lax.fori_loop(0, _C, body, 0)
        plsc.subcore_barrier()
        pltpu.sync_copy(deg_sh.at[pl.ds(s * _BAND, _BAND)],
                        out_hbm.at[c, pl.ds(s * _BAND, _BAND)])

    return k(dst3, ones_col, zeros_col)


def _scatter_pass(h2, src3, dst3, zeros_nd):
    @functools.partial(
        pl.kernel,
        out_type=jax.ShapeDtypeStruct((_NC, _NPAD, _D), jnp.float32),
        mesh=_sc_mesh(),
        scratch_types=[
            pltpu.VMEM((2, _WCH, _K), jnp.int32),
            pltpu.VMEM((2, _WCH, _K), jnp.int32),
            pltpu.VMEM((2, _K, _D), jnp.float32),
            pltpu.VMEM_SHARED((_NPAD, _D), jnp.float32),
            pltpu.SemaphoreType.DMA,
            pltpu.SemaphoreType.DMA,
        ],
    )
    def k(h2_hbm, src_hbm, dst_hbm, zeros_hbm, out_hbm,
          sidx_v, didx_v, rows_v, acc_sh, sem0, sem1):
        c = lax.axis_index("c")
        s = lax.axis_index("s")
        wid = c * _NS + s
        pltpu.sync_copy(zeros_hbm.at[pl.ds(s * _BAND, _BAND)],
                        acc_sh.at[pl.ds(s * _BAND, _BAND)])
        pltpu.sync_copy(src_hbm.at[wid, 0], sidx_v.at[0])
        pltpu.sync_copy(dst_hbm.at[wid, 0], didx_v.at[0])
        plsc.subcore_barrier()

        pltpu.async_copy(h2_hbm.at[sidx_v.at[0, 0]], rows_v.at[0], sem0)

        def win(w, carry):
            wb = w % 2
            nwb = (w + 1) % 2

            @pl.when(w + 1 < _NWIN)
            def _():
                pltpu.sync_copy(src_hbm.at[wid, w + 1], sidx_v.at[nwb])
                pltpu.sync_copy(dst_hbm.at[wid, w + 1], didx_v.at[nwb])

            def pair(p, carry2):
                k0 = 2 * p
                j = w * _WCH + k0
                pltpu.async_copy(h2_hbm.at[sidx_v.at[wb, k0 + 1]],
                                 rows_v.at[1], sem1)
                pltpu.make_async_copy(h2_hbm.at[sidx_v.at[wb, k0]],
                                      rows_v.at[0], sem0).wait()
                pltpu.sync_copy(rows_v.at[0], acc_sh.at[didx_v.at[wb, k0]],
                                add=True)

                @pl.when(j + 2 < _C)
                def _():
                    sel_w = jnp.where(k0 + 2 == _WCH, nwb, wb)
                    sel_k = (k0 + 2) % _WCH
                    pltpu.async_copy(h2_hbm.at[sidx_v.at[sel_w, sel_k]],
                                     rows_v.at[0], sem0)

                pltpu.make_async_copy(h2_hbm.at[sidx_v.at[wb, k0 + 1]],
                                      rows_v.at[1], sem1).wait()
                pltpu.sync_copy(rows_v.at[1],
                                acc_sh.at[didx_v.at[wb, k0 + 1]], add=True)
                return carry2

            lax.fori_loop(0, _WCH // 2, pair, carry)
            return carry

        lax.fori_loop(0, _NWIN, win, 0)
        plsc.subcore_barrier()
        pltpu.sync_copy(acc_sh.at[pl.ds(s * _BAND, _BAND)],
                        out_hbm.at[c, pl.ds(s * _BAND, _BAND)])

    return k(h2, src3, dst3, zeros_nd)


def _tc_a(x, W, gamma2, var2, d0, d1):
    def f(x_ref, w_ref, g_ref, v_ref, d0_ref, d1_ref, h2_ref, dinv_ref):
        sca = g_ref[...] * lax.rsqrt(v_ref[...] + _EPS)
        h = jnp.dot(x_ref[...], w_ref[...] * sca,
                    preferred_element_type=jnp.float32)
        dinv = lax.rsqrt(d0_ref[...] + d1_ref[...] + 1.0)
        h2_ref[...] = h * dinv
        dinv_ref[...] = dinv

    row = lambda i: (i, 0)
    rep = lambda i: (0, 0)
    return pl.pallas_call(
        f,
        grid=(_G,),
        in_specs=[
            pl.BlockSpec((_RB, _D), row),
            pl.BlockSpec((_D, _D), rep),
            pl.BlockSpec((1, _D), rep),
            pl.BlockSpec((1, _D), rep),
            pl.BlockSpec((_RB, 1), row),
            pl.BlockSpec((_RB, 1), row),
        ],
        out_specs=[
            pl.BlockSpec((_RB, _D), row),
            pl.BlockSpec((_RB, 1), row),
        ],
        out_shape=[
            jax.ShapeDtypeStruct((_N, _D), jnp.float32),
            jax.ShapeDtypeStruct((_N, 1), jnp.float32),
        ],
    )(x, W, gamma2, var2, d0, d1)


def _tc_b(p0, p1, h2, dinv, b2, mean2, beta2, gamma2, var2):
    def f(p0_ref, p1_ref, h2_ref, dinv_ref, b_ref, m_ref, bt_ref,
          g_ref, v_ref, out_ref):
        sca = g_ref[...] * lax.rsqrt(v_ref[...] + _EPS)
        t = (b_ref[...] - m_ref[...]) * sca + bt_ref[...]
        tot = p0_ref[...] + p1_ref[...] + h2_ref[...]
        out_ref[...] = jnp.maximum(tot * dinv_ref[...] + t, 0.0)

    row = lambda i: (i, 0)
    rep = lambda i: (0, 0)
    return pl.pallas_call(
        f,
        grid=(_G,),
        in_specs=[
            pl.BlockSpec((_RB, _D), row),
            pl.BlockSpec((_RB, _D), row),
            pl.BlockSpec((_RB, _D), row),
            pl.BlockSpec((_RB, 1), row),
            pl.BlockSpec((1, _D), rep),
            pl.BlockSpec((1, _D), rep),
            pl.BlockSpec((1, _D), rep),
            pl.BlockSpec((1, _D), rep),
            pl.BlockSpec((1, _D), rep),
        ],
        out_specs=pl.BlockSpec((_RB, _D), row),
        out_shape=jax.ShapeDtypeStruct((_N, _D), jnp.float32),
    )(p0, p1, h2, dinv, b2, mean2, beta2, gamma2, var2)


def kernel(x, edge_index, W, b, gamma, beta, running_mean, running_var):
    src3 = edge_index[0].astype(jnp.int32).reshape(_NW, _C, _K)
    dst3 = edge_index[1].astype(jnp.int32).reshape(_NW, _C, _K)
    src4 = src3.reshape(_NW, _NWIN, _WCH, _K)
    dst4 = dst3.reshape(_NW, _NWIN, _WCH, _K)
    ones_col = jnp.ones((_K, _WD), jnp.float32)
    zeros_nd = jnp.zeros((_NPAD, _D), jnp.float32)

    g2 = gamma.reshape(1, _D)
    v2 = running_var.reshape(1, _D)
    b2 = b.reshape(1, _D)
    m2 = running_mean.reshape(1, _D)
    bt2 = beta.reshape(1, _D)

    degp = _deg_pass(dst3, ones_col, zeros_nd)
    d0 = degp[0, :_N, 0:1]
    d1 = degp[1, :_N, 0:1]
    h2, dinv = _tc_a(x, W, g2, v2, d0, d1)
    acc = _scatter_pass(h2, src4, dst4, zeros_nd)
    p0 = acc[0, :_N]
    p1 = acc[1, :_N]
    return _tc_b(p0, p1, h2, dinv, b2, m2, bt2, g2, v2)
